# Initial kernel scaffold; baseline (speedup 1.0000x reference)
import jax
import jax.numpy as jnp
from jax.experimental import pallas as pl


def kernel(feat, W_ctr, W_pre, W_suc, W_left, W_right, W_ctr2, gn1_gamma, gn1_beta, gn2_gamma, gn2_beta, pre_u, pre_v, suc_u, suc_v, left_u, left_v, right_u, right_v):
    raise NotImplementedError("write your pallas kernel here")


# speedup vs baseline: 2.4542x; 2.4542x over previous
import functools

import jax
import jax.numpy as jnp
from jax import lax
from jax.experimental import pallas as pl
from jax.experimental.pallas import tpu as pltpu
from jax.experimental.pallas import tpu_sc as plsc

N = 10000
D = 128
NSC = 6
NREL = 14
NWORK = 32
ET_PAD = 131072
PER_W = ET_PAD // NWORK
CH = 128
NCHUNK = PER_W // CH
ACC_ROWS = 10112
RPT = ACC_ROWS // 16
OPT = 632
BN = 2000


def _relmm(feat, w_all):
    def body(f_ref, w_ref, o_ref):
        f = f_ref[...]
        for r in range(NREL):
            o_ref[r] = lax.dot_general(
                f, w_ref[r], (((1,), (1,)), ((), ())),
                preferred_element_type=jnp.float32,
                precision=lax.Precision.HIGHEST)
    return pl.pallas_call(
        body,
        grid=(N // BN,),
        in_specs=[
            pl.BlockSpec((BN, D), lambda n: (n, 0)),
            pl.BlockSpec((NREL, D, D), lambda n: (0, 0, 0)),
        ],
        out_specs=pl.BlockSpec((NREL, BN, D), lambda n: (0, n, 0)),
        out_shape=jax.ShapeDtypeStruct((NREL, N, D), jnp.float32),
    )(feat, w_all)


def _sc_edge_scatter(x_flat, v_idx, u_idx):
    mesh = plsc.VectorSubcoreMesh(core_axis_name="c", subcore_axis_name="s")

    @functools.partial(
        pl.kernel,
        out_type=jax.ShapeDtypeStruct((2 * N, D), jnp.float32),
        mesh=mesh,
        scratch_types=[
            pltpu.VMEM((CH,), jnp.int32),
            pltpu.VMEM((CH,), jnp.int32),
            pltpu.VMEM((CH, D), jnp.float32),
            pltpu.VMEM_SHARED((ACC_ROWS, D), jnp.float32),
            pltpu.SemaphoreType.DMA,
        ],
    )
    def k(x_hbm, v_hbm, u_hbm, out_hbm, v_buf, u_buf, rows, accum, sem):
        cid = lax.axis_index("c")
        sid = lax.axis_index("s")

        zeros16 = jnp.zeros((16,), jnp.float32)

        @pl.loop(0, CH)
        def _(i):
            for kk in range(D // 16):
                rows[i, pl.ds(kk * 16, 16)] = zeros16

        base_r = pl.multiple_of(sid * RPT, 8)
        off = 0
        for nrow in (CH, CH, CH, CH, RPT - 4 * CH):
            pltpu.sync_copy(rows.at[pl.ds(0, nrow)],
                            accum.at[pl.ds(base_r + off, nrow)])
            off += nrow
        plsc.subcore_barrier()

        base_e = (cid * 16 + sid) * PER_W

        @pl.loop(0, NCHUNK)
        def _(j):
            eoff = pl.multiple_of(base_e + j * CH, CH)
            pltpu.sync_copy(v_hbm.at[pl.ds(eoff, CH)], v_buf)
            pltpu.sync_copy(u_hbm.at[pl.ds(eoff, CH)], u_buf)
            pltpu.async_copy(x_hbm.at[v_buf], rows, sem).wait()
            pltpu.sync_copy(rows, accum.at[u_buf], add=True)

        plsc.subcore_barrier()
        ob = pl.multiple_of(sid * OPT, 8)
        obase = pl.multiple_of(cid * N, 8)

        @pl.when(sid < 15)
        def _():
            pltpu.sync_copy(accum.at[pl.ds(ob, OPT)],
                            out_hbm.at[pl.ds(obase + ob, OPT)])

        @pl.when(sid == 15)
        def _():
            pltpu.sync_copy(accum.at[pl.ds(15 * OPT, N - 15 * OPT)],
                            out_hbm.at[pl.ds(obase + 15 * OPT, N - 15 * OPT)])

    return k(x_flat, v_idx, u_idx)


def _block_tail(feat, t01, w_ctr, w_ctr2, g1, b1, g2, b2):
    def body(f_ref, t0_ref, t1_ref, wc_ref, wc2_ref,
             g1_ref, b1_ref, g2_ref, b2_ref, o_ref):
        f = f_ref[...]
        temp = lax.dot_general(
            f, wc_ref[...], (((1,), (1,)), ((), ())),
            preferred_element_type=jnp.float32,
            precision=lax.Precision.HIGHEST)
        temp = temp + t0_ref[...] + t1_ref[...]
        m = jnp.mean(temp, axis=-1, keepdims=True)
        v = jnp.mean(jnp.square(temp - m), axis=-1, keepdims=True)
        h = (temp - m) * lax.rsqrt(v + 1e-5) * g1_ref[...] + b1_ref[...]
        h = jnp.maximum(h, 0.0)
        h2 = lax.dot_general(
            h, wc2_ref[...], (((1,), (1,)), ((), ())),
            preferred_element_type=jnp.float32,
            precision=lax.Precision.HIGHEST)
        m2 = jnp.mean(h2, axis=-1, keepdims=True)
        v2 = jnp.mean(jnp.square(h2 - m2), axis=-1, keepdims=True)
        n2 = (h2 - m2) * lax.rsqrt(v2 + 1e-5) * g2_ref[...] + b2_ref[...]
        o_ref[...] = jnp.maximum(n2 + f, 0.0)

    nb = N // BN
    row_spec = pl.BlockSpec((BN, D), lambda n: (n, 0))
    full_mat = pl.BlockSpec((D, D), lambda n: (0, 0))
    full_vec = pl.BlockSpec((1, D), lambda n: (0, 0))
    return pl.pallas_call(
        body,
        grid=(nb,),
        in_specs=[
            row_spec,
            pl.BlockSpec((BN, D), lambda n: (n, 0)),
            pl.BlockSpec((BN, D), lambda n: (n + nb, 0)),
            full_mat, full_mat,
            full_vec, full_vec, full_vec, full_vec,
        ],
        out_specs=row_spec,
        out_shape=jax.ShapeDtypeStruct((N, D), jnp.float32),
    )(feat, t01, t01, w_ctr, w_ctr2, g1, b1, g2, b2)


def kernel(feat, W_ctr, W_pre, W_suc, W_left, W_right, W_ctr2,
           gn1_gamma, gn1_beta, gn2_gamma, gn2_beta,
           pre_u, pre_v, suc_u, suc_v, left_u, left_v, right_u, right_v):
    offs = (jnp.arange(NSC, dtype=jnp.int32) * N)[:, None]
    v_idx = jnp.concatenate([
        (pre_v.astype(jnp.int32) + offs).reshape(-1),
        (suc_v.astype(jnp.int32) + offs + NSC * N).reshape(-1),
        left_v.astype(jnp.int32) + 2 * NSC * N,
        right_v.astype(jnp.int32) + (2 * NSC + 1) * N,
    ])
    u_idx = jnp.concatenate([
        pre_u.reshape(-1), suc_u.reshape(-1), left_u, right_u,
    ]).astype(jnp.int32)
    npad = ET_PAD - v_idx.shape[0]
    v_idx = jnp.concatenate([v_idx, jnp.zeros((npad,), jnp.int32)])
    u_idx = jnp.concatenate([u_idx, jnp.full((npad,), N, jnp.int32)])

    x = feat
    for i in range(4):
        w_all = jnp.concatenate(
            [W_pre[i], W_suc[i], W_left[i][None], W_right[i][None]], axis=0)
        X = _relmm(x, w_all).reshape(NREL * N, D)
        t01 = _sc_edge_scatter(X, v_idx, u_idx)
        x = _block_tail(x, t01, W_ctr[i], W_ctr2[i],
                        gn1_gamma[i][None], gn1_beta[i][None],
                        gn2_gamma[i][None], gn2_beta[i][None])
    return x
